# baseline (speedup 1.0000x reference)
import dataclasses
import functools

import jax
import jax.numpy as jnp
from jax import lax
from jax.experimental import pallas as pl
from jax.experimental.pallas import tpu as pltpu
from jax.experimental.pallas import tpu_sc as plsc

B = 16384
II = 256
IC = 128
DI = 128
NC = 2
NS = 16
NW = NC * NS
BPW = B // NW
CH = 128
NCH = BPW // CH
L = 16
BLK = 2048
NBLK = B // BLK
RPB = BLK // 128
CONCAT = 648


def _emb_body(ids_hbm, emb_hbm, wt_hbm, uvw_hbm, out_hbm,
              wt_v, uvw_v, idx_v, emb_v0, emb_v1, sco_v, gsem0, gsem1):
    c = lax.axis_index("c")
    s = lax.axis_index("s")
    wid = s * NC + c

    pltpu.sync_copy(ids_hbm.at[pl.ds(wid * NCH, NCH)], idx_v)
    emb_bufs = (emb_v0, emb_v1)
    gsems = (gsem0, gsem1)
    gcp = pltpu.async_copy(emb_hbm.at[idx_v.at[0]], emb_v0, gsem0)

    pltpu.sync_copy(wt_hbm, wt_v)
    pltpu.sync_copy(uvw_hbm, uvw_v)

    lanes = lax.iota(jnp.int32, L)
    wregs = []
    for j in range(8):
        acc = None
        for t in range(5):
            u_t = plsc.load_gather(uvw_v, [jnp.full((L,), t, jnp.int32)])
            w_t = plsc.load_gather(
                wt_v, [lanes + (256 + L * j), jnp.full((L,), t, jnp.int32)])
            term = w_t * u_t
            acc = term if acc is None else acc + term
        wregs.append(acc)

    lane0 = lanes == 0

    for ch in range(NCH):
        gcp.wait()
        emb_v = emb_bufs[ch % 2]
        if ch + 1 < NCH:
            gcp = pltpu.async_copy(
                emb_hbm.at[idx_v.at[ch + 1]], emb_bufs[(ch + 1) % 2],
                gsems[(ch + 1) % 2])

        @plsc.parallel_loop(0, CH, 1, unroll=4)
        def _(i, _emb=emb_v, _ch=ch):
            prods = [_emb[i, pl.ds(L * j, L)] * wregs[j] for j in range(8)]
            while len(prods) > 1:
                prods = [prods[k] + prods[k + 1]
                         for k in range(0, len(prods), 2)]
            sc = jnp.sum(prods[0])
            plsc.store_scatter(
                sco_v,
                [jnp.full((L,), _ch, jnp.int32), jnp.full((L,), i, jnp.int32)],
                jnp.full((L,), sc, jnp.float32), mask=lane0)

    pltpu.sync_copy(sco_v, out_hbm.at[pl.ds(wid * NCH, NCH)])


_sc_params = pltpu.CompilerParams()
if "needs_layout_passes" in pltpu.CompilerParams.__dataclass_fields__:
    _sc_params = dataclasses.replace(_sc_params, needs_layout_passes=False)

_emb_score = functools.partial(
    pl.kernel,
    out_type=jax.ShapeDtypeStruct((B // CH, CH), jnp.float32),
    mesh=plsc.VectorSubcoreMesh(core_axis_name="c", subcore_axis_name="s"),
    compiler_params=_sc_params,
    scratch_types=[
        pltpu.VMEM((CONCAT, 5), jnp.float32),
        pltpu.VMEM((5,), jnp.float32),
        pltpu.VMEM((NCH, CH), jnp.int32),
        pltpu.VMEM((CH, DI), jnp.float32),
        pltpu.VMEM((CH, DI), jnp.float32),
        pltpu.VMEM((NCH, CH), jnp.float32),
        pltpu.SemaphoreType.DMA,
        pltpu.SemaphoreType.DMA,
    ],
)(_emb_body)


def _rank_body(wt_ref, uvw_ref, wi_ref, if_ref, cf_ref, se_ref, ids_ref,
               out_ref, mx_ref, id_ref):
    i = pl.program_id(0)

    w_full = jnp.dot(wt_ref[...], uvw_ref[...],
                     preferred_element_type=jnp.float32)
    v_item = jnp.dot(wi_ref[...], w_full[384:512, :],
                     preferred_element_type=jnp.float32)
    w_c = w_full[512:640, :]

    dims = (((0,), (1,)), ((), ()))
    rows = []
    for r in range(RPB):
        s_r = lax.dot_general(
            v_item, if_ref[128 * r:128 * (r + 1), :], dims,
            preferred_element_type=jnp.float32)
        s_r = s_r + lax.dot_general(
            w_c, cf_ref[128 * r:128 * (r + 1), :], dims,
            preferred_element_type=jnp.float32)
        rows.append(s_r)
    s = jnp.concatenate(rows, axis=0) + se_ref[...]

    @pl.when(i == 0)
    def _():
        mx_ref[0] = jnp.float32(-3.0e38)
        id_ref[0] = jnp.int32(0)

    bm = jnp.max(s)
    lin = (lax.broadcasted_iota(jnp.int32, s.shape, 0) * 128
           + lax.broadcasted_iota(jnp.int32, s.shape, 1))
    bl = jnp.min(jnp.where(s >= bm, lin, jnp.int32(2 ** 30)))
    bid = jnp.max(jnp.where(lin == bl, ids_ref[...], jnp.int32(-2 ** 31)))

    better = bm > mx_ref[0]
    mx_ref[0] = jnp.where(better, bm, mx_ref[0])
    id_ref[0] = jnp.where(better, bid, id_ref[0])
    out_ref[...] = jnp.full((1, 1), id_ref[0], jnp.int32)


_rank = pl.pallas_call(
    _rank_body,
    grid=(NBLK,),
    in_specs=[
        pl.BlockSpec((CONCAT, 5), lambda i: (0, 0)),
        pl.BlockSpec((5, 1), lambda i: (0, 0)),
        pl.BlockSpec((II, DI), lambda i: (0, 0)),
        pl.BlockSpec((BLK, II), lambda i: (i, 0)),
        pl.BlockSpec((BLK, IC), lambda i: (i, 0)),
        pl.BlockSpec((RPB, 128), lambda i: (i, 0)),
        pl.BlockSpec((RPB, 128), lambda i: (i, 0)),
    ],
    out_specs=pl.BlockSpec((1, 1), lambda i: (0, 0)),
    out_shape=jax.ShapeDtypeStruct((1, 1), jnp.int32),
    scratch_shapes=[pltpu.SMEM((1,), jnp.float32),
                    pltpu.SMEM((1,), jnp.int32)],
)


def kernel(user_id, user_features, item_ids, item_features, cross_features,
           user_emb_table, item_emb_table, pos_emb_table,
           W_user, b_user, W_item, b_item, W_task, b_task,
           user_value_weights):
    ids2d = item_ids.astype(jnp.int32).reshape(B // CH, CH)
    se = _emb_score(ids2d, item_emb_table, W_task, user_value_weights)
    out = _rank(W_task, user_value_weights.reshape(-1, 1), W_item,
                item_features, cross_features, se, ids2d)
    return out.reshape(()).astype(item_ids.dtype)


# speedup vs baseline: 1.1942x; 1.1942x over previous
import dataclasses
import functools

import jax
import jax.numpy as jnp
from jax import lax
from jax.experimental import pallas as pl
from jax.experimental.pallas import tpu as pltpu
from jax.experimental.pallas import tpu_sc as plsc

B = 16384
II = 256
IC = 128
DI = 128
NC = 2
NS = 16
NW = NC * NS
BPW = B // NW
CH = 128
NCH = BPW // CH
L = 16
BLK = 2048
NBLK = B // BLK
RPB = BLK // 128
CONCAT = 648


def _make_emb_body():
    def body(ids_hbm, emb_hbm, wt_hbm, uvw_hbm, out_hbm,
             wt_v, uvw_v, idx_v, emb_v0, emb_v1, sco_v, gsem0, gsem1):
        c = lax.axis_index("c")
        s = lax.axis_index("s")
        wid = s * NC + c

        pltpu.sync_copy(ids_hbm.at[pl.ds(wid * NCH, NCH)], idx_v)
        emb_bufs = (emb_v0, emb_v1)
        gsems = (gsem0, gsem1)
        gcp = pltpu.async_copy(emb_hbm.at[idx_v.at[0]], emb_v0, gsem0)

        pltpu.sync_copy(wt_hbm, wt_v)
        pltpu.sync_copy(uvw_hbm, uvw_v)

        wregs = []
        for j in range(8):
            acc = None
            for t in range(5):
                u_t = plsc.load_gather(uvw_v, [jnp.full((L,), t, jnp.int32)])
                term = wt_v[t, pl.ds(256 + L * j, L)] * u_t
                acc = term if acc is None else acc + term
            wregs.append(acc)

        lane0 = lax.iota(jnp.int32, L) == 0

        for ch in range(NCH):
            gcp.wait()
            emb_v = emb_bufs[ch % 2]
            if ch + 1 < NCH:
                gcp = pltpu.async_copy(
                    emb_hbm.at[idx_v.at[ch + 1]], emb_bufs[(ch + 1) % 2],
                    gsems[(ch + 1) % 2])

            @plsc.parallel_loop(0, CH, 1, unroll=4)
            def _(i, _emb=emb_v, _ch=ch):
                prods = [_emb[i, pl.ds(L * j, L)] * wregs[j]
                         for j in range(8)]
                while len(prods) > 1:
                    prods = [prods[k] + prods[k + 1]
                             for k in range(0, len(prods), 2)]
                sc = jnp.sum(prods[0])
                plsc.store_scatter(
                    sco_v,
                    [jnp.full((L,), _ch, jnp.int32),
                     jnp.full((L,), i, jnp.int32)],
                    jnp.full((L,), sc, jnp.float32), mask=lane0)

        pltpu.sync_copy(sco_v, out_hbm.at[pl.ds(wid * NCH, NCH)])

    return body


_sc_params = pltpu.CompilerParams()
if "needs_layout_passes" in pltpu.CompilerParams.__dataclass_fields__:
    _sc_params = dataclasses.replace(_sc_params, needs_layout_passes=False)

_emb_score = functools.partial(
    pl.kernel,
    out_type=jax.ShapeDtypeStruct((B // CH, CH), jnp.float32),
    mesh=plsc.VectorSubcoreMesh(core_axis_name="c", subcore_axis_name="s"),
    compiler_params=_sc_params,
    scratch_types=[
        pltpu.VMEM((5, CONCAT), jnp.float32),
        pltpu.VMEM((L,), jnp.float32),
        pltpu.VMEM((NCH, CH), jnp.int32),
        pltpu.VMEM((CH, DI), jnp.float32),
        pltpu.VMEM((CH, DI), jnp.float32),
        pltpu.VMEM((NCH, CH), jnp.float32),
        pltpu.SemaphoreType.DMA,
        pltpu.SemaphoreType.DMA,
    ],
)(_make_emb_body())


def _rank_body(wtT_ref, uvw_ref, wiT_ref, if_ref, cf_ref, se_ref, ids_ref,
               out_ref, mx_ref, id_ref):
    i = pl.program_id(0)

    w_full = jnp.dot(uvw_ref[...], wtT_ref[...],
                     preferred_element_type=jnp.float32)
    v_item = jnp.dot(w_full[:, 384:512], wiT_ref[...],
                     preferred_element_type=jnp.float32)
    vw = jnp.concatenate([v_item, w_full[:, 512:640]], axis=1)

    xc = jnp.concatenate([if_ref[...], cf_ref[...]], axis=1)
    dims = (((1,), (1,)), ((), ()))
    rows = [lax.dot_general(vw, xc[128 * r:128 * (r + 1), :], dims,
                            preferred_element_type=jnp.float32)
            for r in range(RPB)]
    s = jnp.concatenate(rows, axis=0) + se_ref[...]

    @pl.when(i == 0)
    def _():
        mx_ref[0] = jnp.float32(-3.0e38)
        id_ref[0] = jnp.int32(0)

    bm = jnp.max(s)
    lin = (lax.broadcasted_iota(jnp.int32, s.shape, 0) * 128
           + lax.broadcasted_iota(jnp.int32, s.shape, 1))
    bl = jnp.min(jnp.where(s >= bm, lin, jnp.int32(2 ** 30)))
    bid = jnp.max(jnp.where(lin == bl, ids_ref[...], jnp.int32(-2 ** 31)))

    better = bm > mx_ref[0]
    mx_ref[0] = jnp.where(better, bm, mx_ref[0])
    id_ref[0] = jnp.where(better, bid, id_ref[0])
    out_ref[...] = jnp.full((1, 1), id_ref[0], jnp.int32)


_rank = pl.pallas_call(
    _rank_body,
    grid=(NBLK,),
    in_specs=[
        pl.BlockSpec((5, CONCAT), lambda i: (0, 0)),
        pl.BlockSpec((1, 5), lambda i: (0, 0)),
        pl.BlockSpec((DI, II), lambda i: (0, 0)),
        pl.BlockSpec((BLK, II), lambda i: (i, 0)),
        pl.BlockSpec((BLK, IC), lambda i: (i, 0)),
        pl.BlockSpec((RPB, 128), lambda i: (i, 0)),
        pl.BlockSpec((RPB, 128), lambda i: (i, 0)),
    ],
    out_specs=pl.BlockSpec((1, 1), lambda i: (0, 0)),
    out_shape=jax.ShapeDtypeStruct((1, 1), jnp.int32),
    scratch_shapes=[pltpu.SMEM((1,), jnp.float32),
                    pltpu.SMEM((1,), jnp.int32)],
)


def kernel(user_id, user_features, item_ids, item_features, cross_features,
           user_emb_table, item_emb_table, pos_emb_table,
           W_user, b_user, W_item, b_item, W_task, b_task,
           user_value_weights):
    ids32 = item_ids.astype(jnp.int32)
    ids2d = ids32.reshape(B // CH, CH)
    wtT = W_task.T
    uvw16 = jnp.pad(user_value_weights, (0, 11))
    se = _emb_score(ids2d, item_emb_table, wtT, uvw16)
    out = _rank(wtT, user_value_weights.reshape(1, -1), W_item.T,
                item_features, cross_features, se, ids2d)
    return out.reshape(()).astype(item_ids.dtype)


# speedup vs baseline: 1.1963x; 1.0017x over previous
import dataclasses
import functools

import jax
import jax.numpy as jnp
from jax import lax
from jax.experimental import pallas as pl
from jax.experimental.pallas import tpu as pltpu
from jax.experimental.pallas import tpu_sc as plsc

B = 16384
II = 256
IC = 128
DI = 128
NC = 2
NS = 16
NW = NC * NS
BPW = B // NW
CH = 128
NCH = BPW // CH
L = 16
BLK = 2048
CONCAT = 648


def _dense_body(wt_ref, uvw_ref, wi_ref, if_ref, cf_ref, out_ref):
    w_full = jnp.dot(wt_ref[...], uvw_ref[...],
                     preferred_element_type=jnp.float32)
    v_item = jnp.dot(wi_ref[...], w_full[384:512, :],
                     preferred_element_type=jnp.float32)
    s = jnp.dot(if_ref[...], v_item, preferred_element_type=jnp.float32)
    s = s + jnp.dot(cf_ref[...], w_full[512:640, :],
                    preferred_element_type=jnp.float32)
    out_ref[...] = s


_dense = pl.pallas_call(
    _dense_body,
    grid=(B // BLK,),
    in_specs=[
        pl.BlockSpec((CONCAT, 5), lambda i: (0, 0)),
        pl.BlockSpec((5, 1), lambda i: (0, 0)),
        pl.BlockSpec((II, DI), lambda i: (0, 0)),
        pl.BlockSpec((BLK, II), lambda i: (i, 0)),
        pl.BlockSpec((BLK, IC), lambda i: (i, 0)),
    ],
    out_specs=pl.BlockSpec((BLK, 1), lambda i: (i, 0)),
    out_shape=jax.ShapeDtypeStruct((B, 1), jnp.float32),
)


def _emb_body(ids_hbm, emb_hbm, wt_hbm, uvw_hbm,
              out_hbm,
              wt_v, uvw_v, idx_v, emb_v, sco_v):
    c = lax.axis_index("c")
    s = lax.axis_index("s")
    wid = s * NC + c
    base = wid * BPW

    pltpu.sync_copy(wt_hbm, wt_v)
    pltpu.sync_copy(uvw_hbm, uvv := uvw_v)
    pltpu.sync_copy(ids_hbm.at[pl.ds(wid * NCH, NCH)], idx_v)

    wregs = []
    for j in range(8):
        acc = None
        for t in range(5):
            u_t = plsc.load_gather(uvv, [jnp.full((L,), t, jnp.int32)])
            term = wt_v[t, pl.ds(256 + L * j, L)] * u_t
            acc = term if acc is None else acc + term
        wregs.append(acc)

    lane0 = lax.iota(jnp.int32, L) == 0

    for ch in range(NCH):
        pltpu.sync_copy(emb_hbm.at[idx_v.at[ch]], emb_v)

        @plsc.parallel_loop(0, CH, 1, unroll=2)
        def _(i, _ch=ch):
            prods = [emb_v[i, pl.ds(L * j, L)] * wregs[j] for j in range(8)]
            while len(prods) > 1:
                prods = [prods[k] + prods[k + 1]
                         for k in range(0, len(prods), 2)]
            sc = jnp.sum(prods[0])
            plsc.store_scatter(sco_v, [jnp.full((L,), _ch * CH + i,
                                                 jnp.int32)],
                               jnp.full((L,), sc, jnp.float32), mask=lane0)

    pltpu.sync_copy(sco_v, out_hbm.at[pl.ds(base, BPW)])


_sc_params = pltpu.CompilerParams()
if "needs_layout_passes" in pltpu.CompilerParams.__dataclass_fields__:
    _sc_params = dataclasses.replace(_sc_params, needs_layout_passes=False)

_emb_score = functools.partial(
    pl.kernel,
    out_type=jax.ShapeDtypeStruct((B,), jnp.float32),
    mesh=plsc.VectorSubcoreMesh(core_axis_name="c", subcore_axis_name="s"),
    compiler_params=_sc_params,
    scratch_types=[
        pltpu.VMEM((5, CONCAT), jnp.float32),
        pltpu.VMEM((L,), jnp.float32),
        pltpu.VMEM((NCH, CH), jnp.int32),
        pltpu.VMEM((CH, DI), jnp.float32),
        pltpu.VMEM((BPW,), jnp.float32),
    ],
)(_emb_body)


def _final_body(sd_ref, se_ref, ids_ref, out_ref):
    sc = sd_ref[...] + se_ref[...]
    m = jnp.max(sc)
    lin = (lax.broadcasted_iota(jnp.int32, sc.shape, 0) * sc.shape[1]
           + lax.broadcasted_iota(jnp.int32, sc.shape, 1))
    sel = jnp.where(sc >= m, lin, jnp.int32(2 ** 30))
    r = jnp.min(sel)
    win = jnp.max(jnp.where(lin == r, ids_ref[...], jnp.int32(-2 ** 31)))
    out_ref[...] = jnp.full((1, 1), win, jnp.int32)


_final = pl.pallas_call(
    _final_body,
    out_shape=jax.ShapeDtypeStruct((1, 1), jnp.int32),
)


def kernel(user_id, user_features, item_ids, item_features, cross_features,
           user_emb_table, item_emb_table, pos_emb_table,
           W_user, b_user, W_item, b_item, W_task, b_task,
           user_value_weights):
    ids32 = item_ids.astype(jnp.int32)
    uvw16 = jnp.pad(user_value_weights, (0, 11))
    s_dense = _dense(W_task, user_value_weights.reshape(-1, 1), W_item,
                     item_features, cross_features)
    s_emb = _emb_score(ids32.reshape(B // CH, CH), item_emb_table,
                       W_task.T, uvw16)
    out = _final(s_dense.reshape(128, 128), s_emb.reshape(128, 128),
                 ids32.reshape(128, 128))
    return out.reshape(()).astype(item_ids.dtype)


# speedup vs baseline: 1.1987x; 1.0020x over previous
import dataclasses
import functools

import jax
import jax.numpy as jnp
from jax import lax
from jax.experimental import pallas as pl
from jax.experimental.pallas import tpu as pltpu
from jax.experimental.pallas import tpu_sc as plsc

B = 16384
II = 256
IC = 128
DI = 128
NC = 2
NS = 16
NW = NC * NS
BPW = B // NW
CH = 128
NCH = BPW // CH
L = 16
BLK = 2048
CONCAT = 648


def _dense_body(wt_ref, uvw_ref, wi_ref, if_ref, cf_ref, out_ref):
    w_full = jnp.dot(wt_ref[...], uvw_ref[...],
                     preferred_element_type=jnp.float32)
    v_item = jnp.dot(wi_ref[...], w_full[384:512, :],
                     preferred_element_type=jnp.float32)
    s = jnp.dot(if_ref[...], v_item, preferred_element_type=jnp.float32)
    s = s + jnp.dot(cf_ref[...], w_full[512:640, :],
                    preferred_element_type=jnp.float32)
    out_ref[...] = s


_dense = pl.pallas_call(
    _dense_body,
    grid=(B // BLK,),
    in_specs=[
        pl.BlockSpec((CONCAT, 5), lambda i: (0, 0)),
        pl.BlockSpec((5, 1), lambda i: (0, 0)),
        pl.BlockSpec((II, DI), lambda i: (0, 0)),
        pl.BlockSpec((BLK, II), lambda i: (i, 0)),
        pl.BlockSpec((BLK, IC), lambda i: (i, 0)),
    ],
    out_specs=pl.BlockSpec((BLK, 1), lambda i: (i, 0)),
    out_shape=jax.ShapeDtypeStruct((B, 1), jnp.float32),
)


def _emb_body(ids_hbm, emb_hbm, wt_hbm, uvw_hbm,
              out_hbm,
              wt_v, uvw_v, idx_v, emb_v0, emb_v1, sco_v, gsem0, gsem1):
    c = lax.axis_index("c")
    s = lax.axis_index("s")
    wid = s * NC + c
    base = wid * BPW

    pltpu.sync_copy(ids_hbm.at[pl.ds(wid * NCH, NCH)], idx_v)
    emb_bufs = (emb_v0, emb_v1)
    gsems = (gsem0, gsem1)
    gcp = pltpu.async_copy(emb_hbm.at[idx_v.at[0]], emb_v0, gsem0)

    pltpu.sync_copy(wt_hbm, wt_v)
    pltpu.sync_copy(uvw_hbm, uvw_v)

    wregs = []
    for j in range(8):
        acc = None
        for t in range(5):
            u_t = plsc.load_gather(uvw_v, [jnp.full((L,), t, jnp.int32)])
            term = wt_v[t, pl.ds(256 + L * j, L)] * u_t
            acc = term if acc is None else acc + term
        wregs.append(acc)

    lane0 = lax.iota(jnp.int32, L) == 0

    for ch in range(NCH):
        gcp.wait()
        emb_v = emb_bufs[ch % 2]
        if ch + 1 < NCH:
            gcp = pltpu.async_copy(
                emb_hbm.at[idx_v.at[ch + 1]], emb_bufs[(ch + 1) % 2],
                gsems[(ch + 1) % 2])

        @plsc.parallel_loop(0, CH, 1, unroll=2)
        def _(i, _ch=ch, _emb=emb_v):
            prods = [_emb[i, pl.ds(L * j, L)] * wregs[j] for j in range(8)]
            while len(prods) > 1:
                prods = [prods[k] + prods[k + 1]
                         for k in range(0, len(prods), 2)]
            sc = jnp.sum(prods[0])
            plsc.store_scatter(sco_v, [jnp.full((L,), _ch * CH + i,
                                                 jnp.int32)],
                               jnp.full((L,), sc, jnp.float32), mask=lane0)

    pltpu.sync_copy(sco_v, out_hbm.at[pl.ds(base, BPW)])


_sc_params = pltpu.CompilerParams()
if "needs_layout_passes" in pltpu.CompilerParams.__dataclass_fields__:
    _sc_params = dataclasses.replace(_sc_params, needs_layout_passes=False)

_emb_score = functools.partial(
    pl.kernel,
    out_type=jax.ShapeDtypeStruct((B,), jnp.float32),
    mesh=plsc.VectorSubcoreMesh(core_axis_name="c", subcore_axis_name="s"),
    compiler_params=_sc_params,
    scratch_types=[
        pltpu.VMEM((5, CONCAT), jnp.float32),
        pltpu.VMEM((L,), jnp.float32),
        pltpu.VMEM((NCH, CH), jnp.int32),
        pltpu.VMEM((CH, DI), jnp.float32),
        pltpu.VMEM((CH, DI), jnp.float32),
        pltpu.VMEM((BPW,), jnp.float32),
        pltpu.SemaphoreType.DMA,
        pltpu.SemaphoreType.DMA,
    ],
)(_emb_body)


def _final_body(sd_ref, se_ref, ids_ref, out_ref):
    sc = sd_ref[...] + se_ref[...]
    m = jnp.max(sc)
    lin = (lax.broadcasted_iota(jnp.int32, sc.shape, 0) * sc.shape[1]
           + lax.broadcasted_iota(jnp.int32, sc.shape, 1))
    sel = jnp.where(sc >= m, lin, jnp.int32(2 ** 30))
    r = jnp.min(sel)
    win = jnp.max(jnp.where(lin == r, ids_ref[...], jnp.int32(-2 ** 31)))
    out_ref[...] = jnp.full((1, 1), win, jnp.int32)


_final = pl.pallas_call(
    _final_body,
    out_shape=jax.ShapeDtypeStruct((1, 1), jnp.int32),
)


def kernel(user_id, user_features, item_ids, item_features, cross_features,
           user_emb_table, item_emb_table, pos_emb_table,
           W_user, b_user, W_item, b_item, W_task, b_task,
           user_value_weights):
    ids32 = item_ids.astype(jnp.int32)
    uvw16 = jnp.pad(user_value_weights, (0, 11))
    s_dense = _dense(W_task, user_value_weights.reshape(-1, 1), W_item,
                     item_features, cross_features)
    s_emb = _emb_score(ids32.reshape(B // CH, CH), item_emb_table,
                       W_task.T, uvw16)
    out = _final(s_dense.reshape(128, 128), s_emb.reshape(128, 128),
                 ids32.reshape(128, 128))
    return out.reshape(()).astype(item_ids.dtype)
